# baseline (speedup 1.0000x reference)
import functools

import jax
import jax.numpy as jnp
from jax import lax
from jax.experimental import pallas as pl
from jax.experimental.pallas import tpu as pltpu
from jax.experimental.pallas import tpu_sc as plsc

D_POS = 128
NUM_CORES = 2
NUM_SUBCORES = 16
NUM_WORKERS = NUM_CORES * NUM_SUBCORES
CHUNK = 128
NBUF = 4


@functools.partial(jax.jit, static_argnames=("total_rows",))
def _sc_gather(pos2d, pe, *, total_rows):
    rows_per_worker = total_rows // NUM_WORKERS
    cpw = rows_per_worker // CHUNK
    mesh = plsc.VectorSubcoreMesh(
        core_axis_name="c",
        subcore_axis_name="s",
        num_cores=NUM_CORES,
        num_subcores=NUM_SUBCORES,
    )

    @functools.partial(
        pl.kernel,
        out_type=jax.ShapeDtypeStruct((total_rows, D_POS), jnp.float32),
        mesh=mesh,
        scratch_types=[
            pltpu.VMEM((cpw, CHUNK), jnp.int32),
            pltpu.VMEM((NBUF, CHUNK, D_POS), jnp.float32),
            pltpu.VMEM_SHARED((pe.shape[0], D_POS), jnp.float32),
            pltpu.SemaphoreType.DMA,
            pltpu.SemaphoreType.DMA,
        ],
    )
    def gather_kernel(pos_hbm, pe_hbm, out_hbm, idx_all, rows_v, pe_sh, gsem, ssem):
        c = lax.axis_index("c")
        s = lax.axis_index("s")
        wid = s * NUM_CORES + c
        base = wid * rows_per_worker

        @pl.when(s == 0)
        def _stage_pe():
            pltpu.sync_copy(pe_hbm, pe_sh)

        plsc.subcore_barrier()

        pltpu.sync_copy(pos_hbm.at[pl.ds(wid * cpw, cpw), :], idx_all)

        def start_gather(g, b, src):
            pltpu.async_copy(src.at[idx_all.at[g]], rows_v.at[b], gsem)

        def wait_gather(b):
            pltpu.make_async_copy(
                out_hbm.at[pl.ds(0, CHUNK), :], rows_v.at[b], gsem
            ).wait()

        def start_store(g, b):
            pltpu.async_copy(
                rows_v.at[b], out_hbm.at[pl.ds(base + g * CHUNK, CHUNK), :], ssem
            )

        def wait_store(b):
            pltpu.make_async_copy(
                rows_v.at[b], out_hbm.at[pl.ds(0, CHUNK), :], ssem
            ).wait()

        start_gather(0, 0, pe_hbm)

        def body(q, carry):
            for b in range(NBUF):
                p = q * NBUF + b
                nb = (b + 1) % NBUF

                @pl.when(p + 1 - NBUF >= 0)
                def _():
                    wait_store(nb)

                @pl.when(p + 1 < cpw)
                def _():
                    start_gather(p + 1, nb, pe_hbm)

                wait_gather(b)
                start_store(p, b)
            return carry

        lax.fori_loop(0, cpw // NBUF, body, 0, unroll=False)

        for _ in range(NBUF - 1):
            wait_store(0)

    return gather_kernel(pos2d, pe)


def kernel(pos, pe):
    batch, seq = pos.shape
    total_rows = batch * seq
    pos2d = pos.reshape(total_rows // CHUNK, CHUNK).astype(jnp.int32)
    out = _sc_gather(pos2d, pe, total_rows=total_rows)
    return out.reshape(batch, seq, D_POS)


# speedup vs baseline: 3.5030x; 3.5030x over previous
import functools

import jax
import jax.numpy as jnp
from jax import lax
from jax.experimental import pallas as pl
from jax.experimental.pallas import tpu as pltpu
from jax.experimental.pallas import tpu_sc as plsc

D_POS = 128
NUM_CORES = 2
NUM_SUBCORES = 16
NUM_WORKERS = NUM_CORES * NUM_SUBCORES
CHUNK = 128
NBUF = 4


@functools.partial(jax.jit, static_argnames=("total_rows",))
def _sc_gather(pos2d, pe, *, total_rows):
    rows_per_worker = total_rows // NUM_WORKERS
    cpw = rows_per_worker // CHUNK
    mesh = plsc.VectorSubcoreMesh(
        core_axis_name="c",
        subcore_axis_name="s",
        num_cores=NUM_CORES,
        num_subcores=NUM_SUBCORES,
    )

    @functools.partial(
        pl.kernel,
        out_type=jax.ShapeDtypeStruct((total_rows, D_POS), jnp.float32),
        mesh=mesh,
        scratch_types=[
            pltpu.VMEM((cpw, CHUNK), jnp.int32),
            pltpu.VMEM((NBUF, CHUNK, D_POS), jnp.float32),
            pltpu.VMEM_SHARED((pe.shape[0], D_POS), jnp.float32),
            pltpu.SemaphoreType.DMA,
            pltpu.SemaphoreType.DMA,
        ],
    )
    def gather_kernel(pos_hbm, pe_hbm, out_hbm, idx_all, rows_v, pe_sh, gsem, ssem):
        c = lax.axis_index("c")
        s = lax.axis_index("s")
        wid = s * NUM_CORES + c
        base = wid * rows_per_worker

        @pl.when(s == 0)
        def _stage_pe():
            pltpu.sync_copy(pe_hbm, pe_sh)

        plsc.subcore_barrier()

        pltpu.sync_copy(pos_hbm.at[pl.ds(wid * cpw, cpw), :], idx_all)

        def start_gather(g, b, src):
            pltpu.async_copy(src.at[idx_all.at[g]], rows_v.at[b], gsem)

        def wait_gather(b):
            pltpu.make_async_copy(
                out_hbm.at[pl.ds(0, CHUNK), :], rows_v.at[b], gsem
            ).wait()

        def start_store(g, b):
            pltpu.async_copy(
                rows_v.at[b], out_hbm.at[pl.ds(base + g * CHUNK, CHUNK), :], ssem
            )

        def wait_store(b):
            pltpu.make_async_copy(
                rows_v.at[b], out_hbm.at[pl.ds(0, CHUNK), :], ssem
            ).wait()

        start_gather(0, 0, pe_sh)

        def body(q, carry):
            for b in range(NBUF):
                p = q * NBUF + b
                nb = (b + 1) % NBUF

                @pl.when(p + 1 - NBUF >= 0)
                def _():
                    wait_store(nb)

                @pl.when(p + 1 < cpw)
                def _():
                    start_gather(p + 1, nb, pe_sh)

                wait_gather(b)
                start_store(p, b)
            return carry

        lax.fori_loop(0, cpw // NBUF, body, 0, unroll=False)

        for _ in range(NBUF - 1):
            wait_store(0)

    return gather_kernel(pos2d, pe)


def kernel(pos, pe):
    batch, seq = pos.shape
    total_rows = batch * seq
    pos2d = pos.reshape(total_rows // CHUNK, CHUNK).astype(jnp.int32)
    out = _sc_gather(pos2d, pe, total_rows=total_rows)
    return out.reshape(batch, seq, D_POS)
